# Initial kernel scaffold; baseline (speedup 1.0000x reference)
import jax
import jax.numpy as jnp
from jax.experimental import pallas as pl


def kernel(images, gt_boxes, gt_classes):
    raise NotImplementedError("write your pallas kernel here")


# speedup vs baseline: 32.8671x; 32.8671x over previous
import functools
import math

import jax
import jax.numpy as jnp
import numpy as np
from jax import lax
from jax.experimental import pallas as pl
from jax.experimental.pallas import tpu as pltpu
from jax.experimental.pallas import tpu_sc as plsc

_MIN_LEVEL = 3
_MAX_LEVEL = 7
_NUM_SCALES = 3
_ASPECT_RATIOS = (0.5, 1.0, 2.0)
_ANCHOR_SIZE = 4.0
_POS_T = 0.5
_NEG_T = 0.4
_EPS = 1e-8

_NC = 2
_NS = 16
_NW = _NC * _NS
_LANES = 16
_C = 4


def _gen_anchors(height, width):
    all_anchors = []
    for level in range(_MIN_LEVEL, _MAX_LEVEL + 1):
        stride = 2 ** level
        fh = int(math.ceil(height / stride))
        fw = int(math.ceil(width / stride))
        cx = (np.arange(fw, dtype=np.float32) + 0.5) * stride
        cy = (np.arange(fh, dtype=np.float32) + 0.5) * stride
        cxg, cyg = np.meshgrid(cx, cy)
        whs = []
        for scale in range(_NUM_SCALES):
            for ar in _ASPECT_RATIOS:
                size = _ANCHOR_SIZE * stride * 2.0 ** (scale / _NUM_SCALES)
                whs.append((size * math.sqrt(ar), size / math.sqrt(ar)))
        whs = np.array(whs, dtype=np.float32)
        centers = np.stack([cxg, cyg], axis=-1).reshape(-1, 1, 2)
        half = whs.reshape(1, -1, 2) / 2.0
        x1y1 = centers - half
        x2y2 = centers + half
        boxes = np.concatenate([x1y1, x2y2], axis=-1).reshape(-1, 4)
        all_anchors.append(boxes)
    return np.concatenate(all_anchors, axis=0)


@functools.lru_cache(maxsize=4)
def _anchor_planes(height, width):
    a = _gen_anchors(height, width)
    m = a.shape[0]
    mpad = -(-m // (_NW * _LANES)) * (_NW * _LANES)
    pad = np.zeros((mpad - m, 4), dtype=np.float32)
    a = np.concatenate([a, pad], axis=0)
    ax1, ay1, ax2, ay2 = a[:, 0], a[:, 1], a[:, 2], a[:, 3]
    one = np.float32(1.0)
    aarea = (ax2 - ax1) * (ay2 - ay1)
    acy = (ay1 + ay2) * np.float32(0.5)
    acx = (ax1 + ax2) * np.float32(0.5)
    ah = np.maximum(ay2 - ay1, np.float32(_EPS))
    aw = np.maximum(ax2 - ax1, np.float32(_EPS))
    planes = np.stack([
        ax1, ay1, ax2, ay2, aarea,
        one / ah, one / aw,
        acy / ah, acx / aw,
        np.log(ah), np.log(aw),
    ], axis=0).astype(np.float32)
    apt = mpad // _NW
    tiled = np.transpose(planes.reshape(11, _NW, apt), (1, 0, 2))
    tiled = tiled.reshape(_NW, 11 * apt).copy()
    return tiled, m, mpad


def _make_sc_call(B, N, apt, mpad):
    nchunk = apt // (_C * _LANES)
    mesh = plsc.VectorSubcoreMesh(core_axis_name="c", subcore_axis_name="s")

    @functools.partial(
        pl.kernel,
        out_type=(
            jax.ShapeDtypeStruct((B, 4, mpad), jnp.float32),
            jax.ShapeDtypeStruct((B, mpad), jnp.float32),
        ),
        mesh=mesh,
        scratch_types=[
            pltpu.VMEM((11 * apt,), jnp.float32),
            pltpu.VMEM((N * 10 * _LANES,), jnp.float32),
            pltpu.VMEM((4 * apt,), jnp.float32),
            pltpu.VMEM((apt,), jnp.float32),
        ],
    )
    def sc_call(ac_hbm, gt_hbm, box_hbm, cls_hbm, ac_v, gt_v, box_v, cls_v):
        wid = lax.axis_index("s") * _NC + lax.axis_index("c")
        base_m = wid * apt
        pltpu.sync_copy(ac_hbm.at[wid], ac_v)

        def chunk_body(ci, _):
            a0 = ci * (_C * _LANES)
            offs = [a0 + c * _LANES for c in range(_C)]
            ax1 = [ac_v[pl.ds(0 * apt + o, _LANES)] for o in offs]
            ay1 = [ac_v[pl.ds(1 * apt + o, _LANES)] for o in offs]
            ax2 = [ac_v[pl.ds(2 * apt + o, _LANES)] for o in offs]
            ay2 = [ac_v[pl.ds(3 * apt + o, _LANES)] for o in offs]
            aar = [ac_v[pl.ds(4 * apt + o, _LANES)] for o in offs]

            zeros = jnp.zeros((_LANES,), jnp.float32)
            neg1 = jnp.full((_LANES,), -1.0, jnp.float32)
            init = (
                (neg1,) * _C,
                (zeros,) * _C,
                (zeros,) * _C,
                (zeros,) * _C,
                (zeros,) * _C,
                (zeros,) * _C,
            )

            def n_step(n, st):
                bi, cy, cx, lh, lw, cl = (list(x) for x in st)
                gbase = n * (10 * _LANES)
                g = [gt_v[pl.ds(gbase + j * _LANES, _LANES)]
                     for j in range(10)]
                for c in range(_C):
                    ix1 = jnp.maximum(ax1[c], g[0])
                    iy1 = jnp.maximum(ay1[c], g[1])
                    ix2 = jnp.minimum(ax2[c], g[2])
                    iy2 = jnp.minimum(ay2[c], g[3])
                    iw = jnp.maximum(ix2 - ix1, 0.0)
                    ih = jnp.maximum(iy2 - iy1, 0.0)
                    inter = iw * ih
                    union = aar[c] + g[4] - inter
                    iou = jnp.where(union > 0.0,
                                    inter / jnp.maximum(union, _EPS), 0.0)
                    upd = iou > bi[c]
                    bi[c] = jnp.where(upd, iou, bi[c])
                    cy[c] = jnp.where(upd, g[5], cy[c])
                    cx[c] = jnp.where(upd, g[6], cx[c])
                    lh[c] = jnp.where(upd, g[7], lh[c])
                    lw[c] = jnp.where(upd, g[8], lw[c])
                    cl[c] = jnp.where(upd, g[9], cl[c])
                return (tuple(bi), tuple(cy), tuple(cx),
                        tuple(lh), tuple(lw), tuple(cl))

            bi, cy, cx, lh, lw, cl = lax.fori_loop(0, N, n_step, init)

            for c in range(_C):
                o = offs[c]

                def arow(r):
                    return ac_v[pl.ds(r * apt + o, _LANES)]

                ty = cy[c] * arow(5) - arow(7)
                tx = cx[c] * arow(6) - arow(8)
                th = lh[c] - arow(9)
                tw = lw[c] - arow(10)
                pos = bi[c] >= _POS_T
                neg = bi[c] < _NEG_T
                cls_t = jnp.where(pos, cl[c],
                                  jnp.where(neg, -1.0, -2.0))
                for j, v in enumerate((ty, tx, th, tw)):
                    box_v[pl.ds(j * apt + o, _LANES)] = v
                cls_v[pl.ds(o, _LANES)] = cls_t
            return 0

        for b in range(B):
            pltpu.sync_copy(gt_hbm.at[b], gt_v)
            lax.fori_loop(0, nchunk, chunk_body, 0)
            for j in range(4):
                pltpu.sync_copy(box_v.at[pl.ds(j * apt, apt)],
                                box_hbm.at[b, j, pl.ds(base_m, apt)])
            pltpu.sync_copy(cls_v, cls_hbm.at[b, pl.ds(base_m, apt)])

    return sc_call


def kernel(images, gt_boxes, gt_classes):
    B, H, W, _ = images.shape
    N = gt_boxes.shape[1]
    ac_np, m, mpad = _anchor_planes(H, W)
    apt = mpad // _NW
    ac = jnp.asarray(ac_np)

    x1 = gt_boxes[..., 0]
    y1 = gt_boxes[..., 1]
    x2 = gt_boxes[..., 2]
    y2 = gt_boxes[..., 3]
    garea = (x2 - x1) * (y2 - y1)
    bcy = (y1 + y2) * 0.5
    bcx = (x1 + x2) * 0.5
    lbh = jnp.log(jnp.maximum(y2 - y1, _EPS))
    lbw = jnp.log(jnp.maximum(x2 - x1, _EPS))
    clsf = gt_classes.astype(jnp.float32)
    g = jnp.stack([x1, y1, x2, y2, garea, bcy, bcx, lbh, lbw, clsf], axis=-1)
    g16 = jnp.broadcast_to(g[..., None], (B, N, 10, _LANES))
    g16 = g16.astype(jnp.float32).reshape(B, N * 10 * _LANES)

    sc_call = _make_sc_call(B, N, apt, mpad)
    box_pad, cls_pad = sc_call(ac, g16)
    box_pad = jnp.transpose(box_pad, (0, 2, 1))
    return box_pad[:, :m, :], cls_pad[:, :m, None]


# speedup vs baseline: 42.6872x; 1.2988x over previous
import functools
import math

import jax
import jax.numpy as jnp
import numpy as np
from jax import lax
from jax.experimental import pallas as pl
from jax.experimental.pallas import tpu as pltpu
from jax.experimental.pallas import tpu_sc as plsc

_MIN_LEVEL = 3
_MAX_LEVEL = 7
_NUM_SCALES = 3
_ASPECT_RATIOS = (0.5, 1.0, 2.0)
_ANCHOR_SIZE = 4.0
_POS_T = 0.5
_NEG_T = 0.4
_EPS = 1e-8

_NC = 2
_NS = 16
_NW = _NC * _NS
_LANES = 16
_C = 4


def _gen_anchors(height, width):
    all_anchors = []
    for level in range(_MIN_LEVEL, _MAX_LEVEL + 1):
        stride = 2 ** level
        fh = int(math.ceil(height / stride))
        fw = int(math.ceil(width / stride))
        cx = (np.arange(fw, dtype=np.float32) + 0.5) * stride
        cy = (np.arange(fh, dtype=np.float32) + 0.5) * stride
        cxg, cyg = np.meshgrid(cx, cy)
        whs = []
        for scale in range(_NUM_SCALES):
            for ar in _ASPECT_RATIOS:
                size = _ANCHOR_SIZE * stride * 2.0 ** (scale / _NUM_SCALES)
                whs.append((size * math.sqrt(ar), size / math.sqrt(ar)))
        whs = np.array(whs, dtype=np.float32)
        centers = np.stack([cxg, cyg], axis=-1).reshape(-1, 1, 2)
        half = whs.reshape(1, -1, 2) / 2.0
        x1y1 = centers - half
        x2y2 = centers + half
        boxes = np.concatenate([x1y1, x2y2], axis=-1).reshape(-1, 4)
        all_anchors.append(boxes)
    return np.concatenate(all_anchors, axis=0)


@functools.lru_cache(maxsize=4)
def _anchor_planes(height, width):
    a = _gen_anchors(height, width)
    m = a.shape[0]
    mpad = -(-m // (_NW * _LANES)) * (_NW * _LANES)
    pad = np.zeros((mpad - m, 4), dtype=np.float32)
    a = np.concatenate([a, pad], axis=0)
    ax1, ay1, ax2, ay2 = a[:, 0], a[:, 1], a[:, 2], a[:, 3]
    one = np.float32(1.0)
    aarea = (ax2 - ax1) * (ay2 - ay1)
    acy = (ay1 + ay2) * np.float32(0.5)
    acx = (ax1 + ax2) * np.float32(0.5)
    ah = np.maximum(ay2 - ay1, np.float32(_EPS))
    aw = np.maximum(ax2 - ax1, np.float32(_EPS))
    planes = np.stack([
        ax1, ay1, ax2, ay2, aarea,
        one / ah, one / aw,
        acy / ah, acx / aw,
        np.log(ah), np.log(aw),
    ], axis=0).astype(np.float32)
    apt = mpad // _NW
    tiled = np.transpose(planes.reshape(11, _NW, apt), (1, 0, 2))
    tiled = tiled.reshape(_NW, 11 * apt).copy()
    return tiled, m, mpad


def _make_sc_call(B, N, apt, mpad):
    nchunk = apt // (_C * _LANES)
    mesh = plsc.VectorSubcoreMesh(core_axis_name="c", subcore_axis_name="s")

    @functools.partial(
        pl.kernel,
        out_type=(
            jax.ShapeDtypeStruct((B, 4, mpad), jnp.float32),
            jax.ShapeDtypeStruct((B, mpad), jnp.float32),
        ),
        mesh=mesh,
        scratch_types=[
            pltpu.VMEM((11 * apt,), jnp.float32),
            pltpu.VMEM((N * 10 * _LANES,), jnp.float32),
            pltpu.VMEM((4 * apt,), jnp.float32),
            pltpu.VMEM((apt,), jnp.float32),
        ],
    )
    def sc_call(ac_hbm, gt_hbm, box_hbm, cls_hbm, ac_v, gt_v, box_v, cls_v):
        wid = lax.axis_index("s") * _NC + lax.axis_index("c")
        base_m = wid * apt
        pltpu.sync_copy(ac_hbm.at[wid], ac_v)

        def chunk_body(ci, _):
            a0 = ci * (_C * _LANES)
            offs = [a0 + c * _LANES for c in range(_C)]
            ax1 = [ac_v[pl.ds(0 * apt + o, _LANES)] for o in offs]
            ay1 = [ac_v[pl.ds(1 * apt + o, _LANES)] for o in offs]
            ax2 = [ac_v[pl.ds(2 * apt + o, _LANES)] for o in offs]
            ay2 = [ac_v[pl.ds(3 * apt + o, _LANES)] for o in offs]
            aar = [ac_v[pl.ds(4 * apt + o, _LANES)] for o in offs]

            zeros = jnp.zeros((_LANES,), jnp.float32)
            ones = jnp.full((_LANES,), 1.0, jnp.float32)
            init = (
                (zeros,) * _C,
                (ones,) * _C,
                (zeros,) * _C,
                (zeros,) * _C,
                (zeros,) * _C,
                (zeros,) * _C,
                (zeros,) * _C,
            )

            def one_n(n, st):
                bint, bun, cy, cx, lh, lw, cl = st
                gbase = n * (10 * _LANES)
                g = [gt_v[pl.ds(gbase + j * _LANES, _LANES)]
                     for j in range(10)]
                for c in range(_C):
                    ix1 = jnp.maximum(ax1[c], g[0])
                    iy1 = jnp.maximum(ay1[c], g[1])
                    ix2 = jnp.minimum(ax2[c], g[2])
                    iy2 = jnp.minimum(ay2[c], g[3])
                    iw = jnp.maximum(ix2 - ix1, 0.0)
                    ih = jnp.maximum(iy2 - iy1, 0.0)
                    inter = iw * ih
                    union = aar[c] + g[4] - inter
                    upd = inter * bun[c] > bint[c] * union
                    bint[c] = jnp.where(upd, inter, bint[c])
                    bun[c] = jnp.where(upd, union, bun[c])
                    cy[c] = jnp.where(upd, g[5], cy[c])
                    cx[c] = jnp.where(upd, g[6], cx[c])
                    lh[c] = jnp.where(upd, g[7], lh[c])
                    lw[c] = jnp.where(upd, g[8], lw[c])
                    cl[c] = jnp.where(upd, g[9], cl[c])
                return bint, bun, cy, cx, lh, lw, cl

            def n_step(n, st):
                st = tuple(list(x) for x in st)
                st = one_n(n * 2, st)
                st = one_n(n * 2 + 1, st)
                return tuple(tuple(x) for x in st)

            nhalf, nrem = divmod(N, 2)
            st = lax.fori_loop(0, nhalf, n_step, init)
            if nrem:
                st = one_n(N - 1, tuple(list(x) for x in st))
            bint, bun, cy, cx, lh, lw, cl = st

            for c in range(_C):
                o = offs[c]

                def arow(r):
                    return ac_v[pl.ds(r * apt + o, _LANES)]

                ty = cy[c] * arow(5) - arow(7)
                tx = cx[c] * arow(6) - arow(8)
                th = lh[c] - arow(9)
                tw = lw[c] - arow(10)
                pos = bint[c] >= _POS_T * bun[c]
                neg = bint[c] < _NEG_T * bun[c]
                cls_t = jnp.where(pos, cl[c],
                                  jnp.where(neg, -1.0, -2.0))
                for j, v in enumerate((ty, tx, th, tw)):
                    box_v[pl.ds(j * apt + o, _LANES)] = v
                cls_v[pl.ds(o, _LANES)] = cls_t
            return 0

        for b in range(B):
            pltpu.sync_copy(gt_hbm.at[b], gt_v)
            lax.fori_loop(0, nchunk, chunk_body, 0)
            for j in range(4):
                pltpu.sync_copy(box_v.at[pl.ds(j * apt, apt)],
                                box_hbm.at[b, j, pl.ds(base_m, apt)])
            pltpu.sync_copy(cls_v, cls_hbm.at[b, pl.ds(base_m, apt)])

    return sc_call


def kernel(images, gt_boxes, gt_classes):
    B, H, W, _ = images.shape
    N = gt_boxes.shape[1]
    ac_np, m, mpad = _anchor_planes(H, W)
    apt = mpad // _NW
    ac = jnp.asarray(ac_np)

    x1 = gt_boxes[..., 0]
    y1 = gt_boxes[..., 1]
    x2 = gt_boxes[..., 2]
    y2 = gt_boxes[..., 3]
    garea = (x2 - x1) * (y2 - y1)
    bcy = (y1 + y2) * 0.5
    bcx = (x1 + x2) * 0.5
    lbh = jnp.log(jnp.maximum(y2 - y1, _EPS))
    lbw = jnp.log(jnp.maximum(x2 - x1, _EPS))
    clsf = gt_classes.astype(jnp.float32)
    g = jnp.stack([x1, y1, x2, y2, garea, bcy, bcx, lbh, lbw, clsf], axis=-1)
    g16 = jnp.broadcast_to(g[..., None], (B, N, 10, _LANES))
    g16 = g16.astype(jnp.float32).reshape(B, N * 10 * _LANES)

    sc_call = _make_sc_call(B, N, apt, mpad)
    box_pad, cls_pad = sc_call(ac, g16)
    box_pad = jnp.transpose(box_pad, (0, 2, 1))
    return box_pad[:, :m, :], cls_pad[:, :m, None]
